# baseline (speedup 1.0000x reference)
import functools

import jax
import jax.numpy as jnp
import numpy as np
from jax import lax
from jax.experimental import pallas as pl
from jax.experimental.pallas import tpu as pltpu
from jax.experimental.pallas import tpu_sc as plsc

_NUM_CLASSES = 16
_IMBALANCE_RATIO = 4.0
_T = 4194304

_WORKERS = 32
_PER_W = _T // _WORKERS
_CH = 8192
_NCH = _PER_W // _CH
_L = 16
_CAP = 40960


def _rank_constants():
    r = _IMBALANCE_RATIO ** (1.0 / (_NUM_CLASSES - 1))
    p = [r**i for i in range(_NUM_CLASSES)]
    s = sum(p)
    p = [x / s for x in p]
    p.reverse()
    cp = np.cumsum(np.asarray(p[:-1], np.float32), dtype=np.float32)
    return [int(v) for v in (cp * np.float32(_T)).astype(np.int32)]

_RANKS = _rank_constants()

_sc_mesh = plsc.VectorSubcoreMesh(core_axis_name="c", subcore_axis_name="s")
_sc_params = pltpu.CompilerParams(needs_layout_passes=False)


def _zero_ref(ref, n):
    z = jnp.zeros((_L,), jnp.int32)

    @plsc.parallel_loop(0, n // _L, unroll=8)
    def body(i):
        ref[pl.ds(i * _L, _L)] = z


@functools.partial(
    pl.kernel,
    out_type=jax.ShapeDtypeStruct((_WORKERS, 65536), jnp.int32),
    mesh=_sc_mesh,
    scratch_types=[pltpu.VMEM((_CH,), jnp.float32),
                   pltpu.VMEM((65536,), jnp.int32)],
    compiler_params=_sc_params,
)
def _hist1(x_hbm, o_hbm, xb, tab):
    wid = lax.axis_index("s") * 2 + lax.axis_index("c")
    _zero_ref(tab, 65536)
    ones = jnp.ones((_L,), jnp.int32)
    base = wid * _PER_W

    def chunk(c, _):
        pltpu.sync_copy(x_hbm.at[pl.ds(base + c * _CH, _CH)], xb)

        @plsc.parallel_loop(0, _CH // _L, unroll=8)
        def vbody(i):
            r = plsc.bitcast(xb[pl.ds(i * _L, _L)], jnp.int32)
            bucket = lax.shift_right_logical(r, 16)
            plsc.addupdate_scatter(tab, [bucket], ones)

        return 0

    lax.fori_loop(0, _NCH, chunk, 0)
    pltpu.sync_copy(tab, o_hbm.at[wid])


@functools.partial(
    pl.kernel,
    out_type=[jax.ShapeDtypeStruct((_WORKERS, 4096), jnp.int32),
              jax.ShapeDtypeStruct((_WORKERS, 16), jnp.int32),
              jax.ShapeDtypeStruct((_WORKERS, _CAP), jnp.int32)],
    mesh=_sc_mesh,
    scratch_types=[pltpu.VMEM((_CH,), jnp.float32),
                   pltpu.VMEM((65536,), jnp.int32),
                   pltpu.VMEM((4096,), jnp.int32),
                   pltpu.VMEM((_CAP + _L,), jnp.int32),
                   pltpu.VMEM((_L,), jnp.int32)],
    compiler_params=_sc_params,
)
def _hist2(x_hbm, lut_hbm, o_hbm, cnt_hbm, cand_hbm, xb, lut, tab, cand,
           cntv):
    wid = lax.axis_index("s") * 2 + lax.axis_index("c")
    pltpu.sync_copy(lut_hbm, lut)
    _zero_ref(tab, 4096)
    ones = jnp.ones((_L,), jnp.int32)
    base = wid * _PER_W

    def chunk(c, cnt):
        pltpu.sync_copy(x_hbm.at[pl.ds(base + c * _CH, _CH)], xb)

        @plsc.parallel_loop(0, _CH // _L, unroll=8, carry=cnt)
        def vbody(i, cnt):
            r = plsc.bitcast(xb[pl.ds(i * _L, _L)], jnp.int32)
            lutv = plsc.load_gather(lut, [lax.shift_right_logical(r, 16)])
            m = lutv > 0
            mono16 = (r ^ lax.shift_right_arithmetic(r, 31)) & 0xFFFF
            w = lutv | mono16
            plsc.addupdate_scatter(tab, [lax.shift_right_logical(w, 8)],
                                   ones, mask=m)
            off = jnp.minimum(cnt, jnp.int32(_CAP))
            plsc.store_compressed(cand.at[pl.ds(off, _L)], w, mask=m)
            return cnt + jnp.sum(m.astype(jnp.int32))

        return vbody

    cnt = lax.fori_loop(0, _NCH, chunk, jnp.int32(0))
    pltpu.sync_copy(tab, o_hbm.at[wid])
    cntv[...] = jnp.full((_L,), cnt, jnp.int32)
    pltpu.sync_copy(cntv, cnt_hbm.at[wid])

    nch = (jnp.minimum(cnt, jnp.int32(_CAP)) + (_CH - 1)) // _CH

    def wout(c, _):
        pltpu.sync_copy(cand.at[pl.ds(c * _CH, _CH)],
                        cand_hbm.at[wid, pl.ds(c * _CH, _CH)])
        return 0

    lax.fori_loop(0, nch, wout, 0)


@functools.partial(
    pl.kernel,
    out_type=jax.ShapeDtypeStruct((_WORKERS, 4096), jnp.int32),
    mesh=_sc_mesh,
    scratch_types=[pltpu.VMEM((_CH,), jnp.int32),
                   pltpu.VMEM((_L,), jnp.int32),
                   pltpu.VMEM((4096,), jnp.int32),
                   pltpu.VMEM((4096,), jnp.int32),
                   pltpu.VMEM((_CH,), jnp.float32),
                   pltpu.VMEM((65536,), jnp.int32)],
    compiler_params=_sc_params,
)
def _hist3(cand_hbm, cnt_hbm, lut2_hbm, x_hbm, lut_hbm, o_hbm,
           cb, cntv, lut2, tab, xb, lut):
    wid = lax.axis_index("s") * 2 + lax.axis_index("c")
    pltpu.sync_copy(lut2_hbm, lut2)
    _zero_ref(tab, 4096)
    pltpu.sync_copy(cnt_hbm.at[wid], cntv)
    cnt = cntv[...][0]
    ones = jnp.ones((_L,), jnp.int32)
    lane = lax.broadcasted_iota(jnp.int32, (_L,), 0)

    def cand_path():
        nch = (cnt + (_CH - 1)) // _CH

        def chunk(c, _):
            pltpu.sync_copy(cand_hbm.at[wid, pl.ds(c * _CH, _CH)], cb)
            cbase = c * _CH

            @plsc.parallel_loop(0, _CH // _L, unroll=8)
            def vbody(i):
                w = cb[pl.ds(i * _L, _L)]
                lm = (cbase + i * _L + lane) < cnt
                v256 = plsc.load_gather(
                    lut2, [lax.shift_right_logical(w, 8) & 0xFFF])
                plsc.addupdate_scatter(tab, [v256 + (w & 0xFF)], ones, mask=lm)

            return 0

        lax.fori_loop(0, nch, chunk, 0)

    def full_path():
        pltpu.sync_copy(lut_hbm, lut)
        base = wid * _PER_W

        def chunk(c, _):
            pltpu.sync_copy(x_hbm.at[pl.ds(base + c * _CH, _CH)], xb)

            @plsc.parallel_loop(0, _CH // _L, unroll=8)
            def vbody(i):
                r = plsc.bitcast(xb[pl.ds(i * _L, _L)], jnp.int32)
                lutv = plsc.load_gather(
                    lut, [lax.shift_right_logical(r, 16)])
                m = lutv > 0
                mono16 = (r ^ lax.shift_right_arithmetic(r, 31)) & 0xFFFF
                w = lutv | mono16
                v256 = plsc.load_gather(
                    lut2, [lax.shift_right_logical(w, 8) & 0xFFF])
                plsc.addupdate_scatter(tab, [v256 + (w & 0xFF)], ones, mask=m)

            return 0

        lax.fori_loop(0, _NCH, chunk, 0)

    lax.cond(cnt <= _CAP, cand_path, full_path)
    pltpu.sync_copy(tab, o_hbm.at[wid])


def _tri_le(n):
    a = lax.broadcasted_iota(jnp.int32, (n, n), 0)
    b = lax.broadcasted_iota(jnp.int32, (n, n), 1)
    return (a <= b).astype(jnp.float32)


def _rev_mat(n):
    a = lax.broadcasted_iota(jnp.int32, (n, n), 0)
    b = lax.broadcasted_iota(jnp.int32, (n, n), 1)
    return (a + b == n - 1).astype(jnp.float32)


def _find1_body(hist_ref, lut_ref, meta_ref):
    hraw = jnp.sum(hist_ref[...].astype(jnp.float32), axis=0)
    neg = jnp.dot(jnp.dot(_rev_mat(256), hraw[256:, :],
                          preferred_element_type=jnp.float32,
                          precision=lax.Precision.HIGHEST),
                  _rev_mat(128), preferred_element_type=jnp.float32,
                  precision=lax.Precision.HIGHEST)
    h = jnp.concatenate([neg, hraw[:256, :]], axis=0)
    rowcum = jnp.dot(h, _tri_le(128), preferred_element_type=jnp.float32,
                     precision=lax.Precision.HIGHEST)
    srow = rowcum[:, 127:128]
    a = lax.broadcasted_iota(jnp.int32, (512, 512), 0)
    b = lax.broadcasted_iota(jnp.int32, (512, 512), 1)
    strict = (b < a).astype(jnp.float32)
    coarse = jnp.dot(strict, srow, preferred_element_type=jnp.float32,
                     precision=lax.Precision.HIGHEST)
    cum = rowcum + coarse

    bs, rs, firsts = [], [], []
    prev_b = None
    for k in _RANKS:
        le = (cum <= float(k)).astype(jnp.float32)
        bj = jnp.sum(le).astype(jnp.int32)
        cumexcl = jnp.max(cum * le)
        rj = jnp.int32(k) - cumexcl.astype(jnp.int32)
        bs.append(bj)
        rs.append(rj)
        firsts.append(jnp.int32(1) if prev_b is None
                      else (bj != prev_b).astype(jnp.int32))
        prev_b = bj
    uplus = []
    acc = jnp.int32(0)
    for f in firsts:
        acc = acc + f
        uplus.append(acc)

    gi = lax.broadcasted_iota(jnp.int32, (512, 128), 0)
    gj = lax.broadcasted_iota(jnp.int32, (512, 128), 1)
    g = gi * 128 + gj
    eq_any = jnp.zeros((512, 128), jnp.bool_)
    lev = jnp.zeros((512, 128), jnp.int32)
    for j in range(15):
        eq_any = eq_any | (g == bs[j])
        lev = lev + firsts[j] * (g >= bs[j]).astype(jnp.int32)
    lut_s = jnp.where(eq_any, lev * 65536, 0)
    flipped = jnp.dot(
        jnp.dot(_rev_mat(256), lut_s[:256, :].astype(jnp.float32),
                preferred_element_type=jnp.float32,
                precision=lax.Precision.HIGHEST),
        _rev_mat(128), preferred_element_type=jnp.float32,
        precision=lax.Precision.HIGHEST).astype(jnp.int32)
    lut_ref[...] = jnp.concatenate([lut_s[256:, :], flipped], axis=0)

    ri = lax.broadcasted_iota(jnp.int32, (8, 128), 0)
    ci = lax.broadcasted_iota(jnp.int32, (8, 128), 1)
    meta = jnp.zeros((8, 128), jnp.int32)
    for j in range(15):
        cj = (ci == j)
        meta = meta + jnp.where((ri == 0) & cj, rs[j], 0)
        meta = meta + jnp.where((ri == 1) & cj, uplus[j], 0)
        meta = meta + jnp.where((ri == 2) & cj, bs[j], 0)
    meta_ref[...] = meta


def _find2_body(hist_ref, meta_ref, lut2_ref, meta2_ref):
    h = jnp.sum(hist_ref[...].astype(jnp.float32), axis=0)
    cum = jnp.dot(h, _tri_le(256), preferred_element_type=jnp.float32,
                  precision=lax.Precision.HIGHEST)
    rowid = lax.broadcasted_iota(jnp.int32, (16, 256), 0)

    ss, r2s, firsts, pairids = [], [], [], []
    prev_pair = None
    for j in range(15):
        rj = meta_ref[0, j]
        uj = meta_ref[1, j]
        rm = (rowid == uj).astype(jnp.float32)
        cumsel = cum * rm
        le = (cumsel <= rj.astype(jnp.float32)).astype(jnp.float32) * rm
        sj = jnp.sum(le).astype(jnp.int32)
        cumexcl = jnp.max(cumsel * le)
        r2j = rj - cumexcl.astype(jnp.int32)
        pairj = uj * 256 + sj
        ss.append(sj)
        r2s.append(r2j)
        pairids.append(pairj)
        firsts.append(jnp.int32(1) if prev_pair is None
                      else (pairj != prev_pair).astype(jnp.int32))
        prev_pair = pairj
    vplus = []
    acc = jnp.int32(0)
    for f in firsts:
        acc = acc + f
        vplus.append(acc)

    colid = lax.broadcasted_iota(jnp.int32, (16, 256), 1)
    g = rowid * 256 + colid
    eq_any = jnp.zeros((16, 256), jnp.bool_)
    lev = jnp.zeros((16, 256), jnp.int32)
    for j in range(15):
        eq_any = eq_any | (g == pairids[j])
        lev = lev + firsts[j] * (g >= pairids[j]).astype(jnp.int32)
    lut2_ref[...] = jnp.where(eq_any, lev * 256, 0)

    ri = lax.broadcasted_iota(jnp.int32, (8, 128), 0)
    ci = lax.broadcasted_iota(jnp.int32, (8, 128), 1)
    meta2 = jnp.zeros((8, 128), jnp.int32)
    for j in range(15):
        cj = (ci == j)
        meta2 = meta2 + jnp.where((ri == 0) & cj, r2s[j], 0)
        meta2 = meta2 + jnp.where((ri == 1) & cj, vplus[j], 0)
        meta2 = meta2 + jnp.where((ri == 2) & cj, meta_ref[2, j], 0)
        meta2 = meta2 + jnp.where((ri == 3) & cj, ss[j], 0)
    meta2_ref[...] = meta2


def _classify_body(x_ref, hist_ref, meta2_ref, o_ref):
    h = jnp.sum(hist_ref[...].astype(jnp.float32), axis=0)
    cum = jnp.dot(h, _tri_le(256), preferred_element_type=jnp.float32,
                  precision=lax.Precision.HIGHEST)
    rowid = lax.broadcasted_iota(jnp.int32, (16, 256), 0)

    xb = x_ref[...]
    acc = jnp.zeros(xb.shape, jnp.int32)
    for j in range(15):
        r2j = meta2_ref[0, j]
        vj = meta2_ref[1, j]
        bj = meta2_ref[2, j]
        sj = meta2_ref[3, j]
        rm = (rowid == vj).astype(jnp.float32)
        cumsel = cum * rm
        le = (cumsel <= r2j.astype(jnp.float32)).astype(jnp.float32) * rm
        fj = jnp.sum(le).astype(jnp.int32)
        keyj = (bj - 32768) * 65536 + sj * 256 + fj
        neg = lax.shift_right_arithmetic(keyj, 31)
        fbits = keyj ^ (neg & 0x7FFFFFFF)
        bndj = lax.bitcast_convert_type(fbits, jnp.float32)
        acc = acc + (xb > bndj).astype(jnp.int32)
    o_ref[...] = acc


def kernel(input):
    x = input
    assert x.shape == (_T,) and x.dtype == jnp.float32

    h1 = _hist1(x)
    lut16, meta = pl.pallas_call(
        _find1_body,
        out_shape=[jax.ShapeDtypeStruct((512, 128), jnp.int32),
                   jax.ShapeDtypeStruct((8, 128), jnp.int32)],
    )(h1.reshape(_WORKERS, 512, 128))

    h2, cnts, cand = _hist2(x, lut16.reshape(65536))
    lut2, meta2 = pl.pallas_call(
        _find2_body,
        out_shape=[jax.ShapeDtypeStruct((16, 256), jnp.int32),
                   jax.ShapeDtypeStruct((8, 128), jnp.int32)],
    )(h2.reshape(_WORKERS, 16, 256), meta)

    h3 = _hist3(cand, cnts, lut2.reshape(4096), x, lut16.reshape(65536))

    rows, cols = 4096, 1024
    block = 512
    classes = pl.pallas_call(
        _classify_body,
        grid=(rows // block,),
        in_specs=[pl.BlockSpec((block, cols), lambda i: (i, 0)),
                  pl.BlockSpec((_WORKERS, 16, 256), lambda i: (0, 0, 0)),
                  pl.BlockSpec((8, 128), lambda i: (0, 0))],
        out_specs=pl.BlockSpec((block, cols), lambda i: (i, 0)),
        out_shape=jax.ShapeDtypeStruct((rows, cols), jnp.int32),
    )(x.reshape(rows, cols), h3.reshape(_WORKERS, 16, 256), meta2)
    return classes.reshape(_T)


# speedup vs baseline: 1.0535x; 1.0535x over previous
import functools

import jax
import jax.numpy as jnp
import numpy as np
from jax import lax
from jax.experimental import pallas as pl
from jax.experimental.pallas import tpu as pltpu
from jax.experimental.pallas import tpu_sc as plsc

_NUM_CLASSES = 16
_IMBALANCE_RATIO = 4.0
_T = 4194304

_WORKERS = 32
_PER_W = _T // _WORKERS
_CH = 8192
_NCH = _PER_W // _CH
_L = 16
_CAP = 40960


def _rank_constants():
    r = _IMBALANCE_RATIO ** (1.0 / (_NUM_CLASSES - 1))
    p = [r**i for i in range(_NUM_CLASSES)]
    s = sum(p)
    p = [x / s for x in p]
    p.reverse()
    cp = np.cumsum(np.asarray(p[:-1], np.float32), dtype=np.float32)
    return [int(v) for v in (cp * np.float32(_T)).astype(np.int32)]

_RANKS = _rank_constants()

_sc_mesh = plsc.VectorSubcoreMesh(core_axis_name="c", subcore_axis_name="s")
_sc_params = pltpu.CompilerParams(needs_layout_passes=False)


def _zero_ref(ref, n):
    z = jnp.zeros((_L,), jnp.int32)

    @plsc.parallel_loop(0, n // _L, unroll=8)
    def body(i):
        ref[pl.ds(i * _L, _L)] = z


@functools.partial(
    pl.kernel,
    out_type=jax.ShapeDtypeStruct((_WORKERS, 65536), jnp.int32),
    mesh=_sc_mesh,
    scratch_types=[pltpu.VMEM((_CH,), jnp.float32),
                   pltpu.VMEM((65536,), jnp.int32)],
    compiler_params=_sc_params,
)
def _hist1(x_hbm, o_hbm, xb, tab):
    wid = lax.axis_index("s") * 2 + lax.axis_index("c")
    _zero_ref(tab, 65536)
    ones = jnp.ones((_L,), jnp.int32)
    base = wid * _PER_W

    def chunk(c, _):
        pltpu.sync_copy(x_hbm.at[pl.ds(base + c * _CH, _CH)], xb)

        @plsc.parallel_loop(0, _CH // _L, unroll=8)
        def vbody(i):
            r = plsc.bitcast(xb[pl.ds(i * _L, _L)], jnp.int32)
            bucket = lax.shift_right_logical(r, 16)
            plsc.addupdate_scatter(tab, [bucket], ones)

        return 0

    lax.fori_loop(0, _NCH, chunk, 0)
    pltpu.sync_copy(tab, o_hbm.at[wid])


@functools.partial(
    pl.kernel,
    out_type=[jax.ShapeDtypeStruct((_WORKERS, 4096), jnp.int32),
              jax.ShapeDtypeStruct((_WORKERS, 16), jnp.int32),
              jax.ShapeDtypeStruct((_WORKERS, _CAP), jnp.int32)],
    mesh=_sc_mesh,
    scratch_types=[pltpu.VMEM((_CH,), jnp.float32),
                   pltpu.VMEM((65536,), jnp.int32),
                   pltpu.VMEM((4096,), jnp.int32),
                   pltpu.VMEM((_CAP + _L,), jnp.int32),
                   pltpu.VMEM((_L,), jnp.int32)],
    compiler_params=_sc_params,
)
def _hist2(x_hbm, lut_hbm, o_hbm, cnt_hbm, cand_hbm, xb, lut, tab, cand,
           cntv):
    wid = lax.axis_index("s") * 2 + lax.axis_index("c")
    pltpu.sync_copy(lut_hbm, lut)
    _zero_ref(tab, 4096)
    ones = jnp.ones((_L,), jnp.int32)
    base = wid * _PER_W

    def chunk(c, cnt):
        pltpu.sync_copy(x_hbm.at[pl.ds(base + c * _CH, _CH)], xb)

        @plsc.parallel_loop(0, _CH // _L, unroll=8, carry=cnt)
        def vbody(i, cnt):
            r = plsc.bitcast(xb[pl.ds(i * _L, _L)], jnp.int32)
            lutv = plsc.load_gather(lut, [lax.shift_right_logical(r, 16)])
            m = lutv > 0
            mono16 = (r ^ lax.shift_right_arithmetic(r, 31)) & 0xFFFF
            w = lutv | mono16
            plsc.addupdate_scatter(tab, [lax.shift_right_logical(w, 8)],
                                   ones, mask=m)
            off = jnp.minimum(cnt, jnp.int32(_CAP))
            plsc.store_compressed(cand.at[pl.ds(off, _L)], w, mask=m)
            return cnt + jnp.sum(m.astype(jnp.int32))

        return vbody

    cnt = lax.fori_loop(0, _NCH, chunk, jnp.int32(0))
    pltpu.sync_copy(tab, o_hbm.at[wid])
    cntv[...] = jnp.full((_L,), cnt, jnp.int32)
    pltpu.sync_copy(cntv, cnt_hbm.at[wid])

    nch = (jnp.minimum(cnt, jnp.int32(_CAP)) + (_CH - 1)) // _CH

    def wout(c, _):
        pltpu.sync_copy(cand.at[pl.ds(c * _CH, _CH)],
                        cand_hbm.at[wid, pl.ds(c * _CH, _CH)])
        return 0

    lax.fori_loop(0, nch, wout, 0)


@functools.partial(
    pl.kernel,
    out_type=jax.ShapeDtypeStruct((_WORKERS, 4096), jnp.int32),
    mesh=_sc_mesh,
    scratch_types=[pltpu.VMEM((_CH,), jnp.int32),
                   pltpu.VMEM((_L,), jnp.int32),
                   pltpu.VMEM((4096,), jnp.int32),
                   pltpu.VMEM((4096,), jnp.int32),
                   pltpu.VMEM((_CH,), jnp.float32),
                   pltpu.VMEM((65536,), jnp.int32)],
    compiler_params=_sc_params,
)
def _hist3(cand_hbm, cnt_hbm, lut2_hbm, x_hbm, lut_hbm, o_hbm,
           cb, cntv, lut2, tab, xb, lut):
    wid = lax.axis_index("s") * 2 + lax.axis_index("c")
    pltpu.sync_copy(lut2_hbm, lut2)
    _zero_ref(tab, 4096)
    pltpu.sync_copy(cnt_hbm.at[wid], cntv)
    cnt = cntv[...][0]
    ones = jnp.ones((_L,), jnp.int32)
    lane = lax.broadcasted_iota(jnp.int32, (_L,), 0)

    def cand_path():
        nch = (cnt + (_CH - 1)) // _CH

        def chunk(c, _):
            pltpu.sync_copy(cand_hbm.at[wid, pl.ds(c * _CH, _CH)], cb)
            cbase = c * _CH

            @plsc.parallel_loop(0, _CH // _L, unroll=8)
            def vbody(i):
                w = cb[pl.ds(i * _L, _L)]
                lm = (cbase + i * _L + lane) < cnt
                v256 = plsc.load_gather(
                    lut2, [lax.shift_right_logical(w, 8) & 0xFFF])
                plsc.addupdate_scatter(tab, [v256 + (w & 0xFF)], ones, mask=lm)

            return 0

        lax.fori_loop(0, nch, chunk, 0)

    def full_path():
        pltpu.sync_copy(lut_hbm, lut)
        base = wid * _PER_W

        def chunk(c, _):
            pltpu.sync_copy(x_hbm.at[pl.ds(base + c * _CH, _CH)], xb)

            @plsc.parallel_loop(0, _CH // _L, unroll=8)
            def vbody(i):
                r = plsc.bitcast(xb[pl.ds(i * _L, _L)], jnp.int32)
                lutv = plsc.load_gather(
                    lut, [lax.shift_right_logical(r, 16)])
                m = lutv > 0
                mono16 = (r ^ lax.shift_right_arithmetic(r, 31)) & 0xFFFF
                w = lutv | mono16
                v256 = plsc.load_gather(
                    lut2, [lax.shift_right_logical(w, 8) & 0xFFF])
                plsc.addupdate_scatter(tab, [v256 + (w & 0xFF)], ones, mask=m)

            return 0

        lax.fori_loop(0, _NCH, chunk, 0)

    lax.cond(cnt <= _CAP, cand_path, full_path)
    pltpu.sync_copy(tab, o_hbm.at[wid])


def _tri_le(n):
    a = lax.broadcasted_iota(jnp.int32, (n, n), 0)
    b = lax.broadcasted_iota(jnp.int32, (n, n), 1)
    return (a <= b).astype(jnp.float32)


def _rev_mat(n):
    a = lax.broadcasted_iota(jnp.int32, (n, n), 0)
    b = lax.broadcasted_iota(jnp.int32, (n, n), 1)
    return (a + b == n - 1).astype(jnp.float32)


def _find1_body(hist_ref, lut_ref, meta_ref):
    hraw = jnp.sum(hist_ref[...].astype(jnp.float32), axis=0)
    neg = jnp.dot(jnp.dot(_rev_mat(256), hraw[256:, :],
                          preferred_element_type=jnp.float32,
                          precision=lax.Precision.HIGHEST),
                  _rev_mat(128), preferred_element_type=jnp.float32,
                  precision=lax.Precision.HIGHEST)
    h = jnp.concatenate([neg, hraw[:256, :]], axis=0)
    rowcum = jnp.dot(h, _tri_le(128), preferred_element_type=jnp.float32,
                     precision=lax.Precision.HIGHEST)
    srow = rowcum[:, 127:128]
    a = lax.broadcasted_iota(jnp.int32, (512, 512), 0)
    b = lax.broadcasted_iota(jnp.int32, (512, 512), 1)
    strict = (b < a).astype(jnp.float32)
    coarse = jnp.dot(strict, srow, preferred_element_type=jnp.float32,
                     precision=lax.Precision.HIGHEST)
    cum = rowcum + coarse

    bs, rs, firsts = [], [], []
    prev_b = None
    for k in _RANKS:
        le = (cum <= float(k)).astype(jnp.float32)
        bj = jnp.sum(le).astype(jnp.int32)
        cumexcl = jnp.max(cum * le)
        rj = jnp.int32(k) - cumexcl.astype(jnp.int32)
        bs.append(bj)
        rs.append(rj)
        firsts.append(jnp.int32(1) if prev_b is None
                      else (bj != prev_b).astype(jnp.int32))
        prev_b = bj
    uplus = []
    acc = jnp.int32(0)
    for f in firsts:
        acc = acc + f
        uplus.append(acc)

    gi = lax.broadcasted_iota(jnp.int32, (512, 128), 0)
    gj = lax.broadcasted_iota(jnp.int32, (512, 128), 1)
    g = gi * 128 + gj
    eq_any = jnp.zeros((512, 128), jnp.bool_)
    lev = jnp.zeros((512, 128), jnp.int32)
    for j in range(15):
        eq_any = eq_any | (g == bs[j])
        lev = lev + firsts[j] * (g >= bs[j]).astype(jnp.int32)
    lut_s = jnp.where(eq_any, lev * 65536, 0)
    flipped = jnp.dot(
        jnp.dot(_rev_mat(256), lut_s[:256, :].astype(jnp.float32),
                preferred_element_type=jnp.float32,
                precision=lax.Precision.HIGHEST),
        _rev_mat(128), preferred_element_type=jnp.float32,
        precision=lax.Precision.HIGHEST).astype(jnp.int32)
    lut_ref[...] = jnp.concatenate([lut_s[256:, :], flipped], axis=0)

    ri = lax.broadcasted_iota(jnp.int32, (8, 128), 0)
    ci = lax.broadcasted_iota(jnp.int32, (8, 128), 1)
    meta = jnp.zeros((8, 128), jnp.int32)
    for j in range(15):
        cj = (ci == j)
        meta = meta + jnp.where((ri == 0) & cj, rs[j], 0)
        meta = meta + jnp.where((ri == 1) & cj, uplus[j], 0)
        meta = meta + jnp.where((ri == 2) & cj, bs[j], 0)
    meta_ref[...] = meta


def _find2_body(hist_ref, meta_ref, lut2_ref, meta2_ref):
    h = jnp.sum(hist_ref[...].astype(jnp.float32), axis=0)
    cum = jnp.dot(h, _tri_le(256), preferred_element_type=jnp.float32,
                  precision=lax.Precision.HIGHEST)
    rowid = lax.broadcasted_iota(jnp.int32, (16, 256), 0)

    ss, r2s, firsts, pairids = [], [], [], []
    prev_pair = None
    for j in range(15):
        rj = meta_ref[0, j]
        uj = meta_ref[1, j]
        rm = (rowid == uj).astype(jnp.float32)
        cumsel = cum * rm
        le = (cumsel <= rj.astype(jnp.float32)).astype(jnp.float32) * rm
        sj = jnp.sum(le).astype(jnp.int32)
        cumexcl = jnp.max(cumsel * le)
        r2j = rj - cumexcl.astype(jnp.int32)
        pairj = uj * 256 + sj
        ss.append(sj)
        r2s.append(r2j)
        pairids.append(pairj)
        firsts.append(jnp.int32(1) if prev_pair is None
                      else (pairj != prev_pair).astype(jnp.int32))
        prev_pair = pairj
    vplus = []
    acc = jnp.int32(0)
    for f in firsts:
        acc = acc + f
        vplus.append(acc)

    colid = lax.broadcasted_iota(jnp.int32, (16, 256), 1)
    g = rowid * 256 + colid
    eq_any = jnp.zeros((16, 256), jnp.bool_)
    lev = jnp.zeros((16, 256), jnp.int32)
    for j in range(15):
        eq_any = eq_any | (g == pairids[j])
        lev = lev + firsts[j] * (g >= pairids[j]).astype(jnp.int32)
    lut2_ref[...] = jnp.where(eq_any, lev * 256, 0)

    ri = lax.broadcasted_iota(jnp.int32, (8, 128), 0)
    ci = lax.broadcasted_iota(jnp.int32, (8, 128), 1)
    meta2 = jnp.zeros((8, 128), jnp.int32)
    for j in range(15):
        cj = (ci == j)
        meta2 = meta2 + jnp.where((ri == 0) & cj, r2s[j], 0)
        meta2 = meta2 + jnp.where((ri == 1) & cj, vplus[j], 0)
        meta2 = meta2 + jnp.where((ri == 2) & cj, meta_ref[2, j], 0)
        meta2 = meta2 + jnp.where((ri == 3) & cj, ss[j], 0)
    meta2_ref[...] = meta2


def _find3_body(hist_ref, meta2_ref, bnd_ref):
    h = jnp.sum(hist_ref[...].astype(jnp.float32), axis=0)
    cum = jnp.dot(h, _tri_le(256), preferred_element_type=jnp.float32,
                  precision=lax.Precision.HIGHEST)
    rowid = lax.broadcasted_iota(jnp.int32, (16, 256), 0)

    ci = lax.broadcasted_iota(jnp.int32, (1, 128), 1)
    keyvec = jnp.zeros((1, 128), jnp.int32)
    for j in range(15):
        r2j = meta2_ref[0, j]
        vj = meta2_ref[1, j]
        bj = meta2_ref[2, j]
        sj = meta2_ref[3, j]
        rm = (rowid == vj).astype(jnp.float32)
        cumsel = cum * rm
        le = (cumsel <= r2j.astype(jnp.float32)).astype(jnp.float32) * rm
        fj = jnp.sum(le).astype(jnp.int32)
        keyj = (bj - 32768) * 65536 + sj * 256 + fj
        keyvec = keyvec + jnp.where(ci == j, keyj, 0)
    neg = lax.shift_right_arithmetic(keyvec, 31)
    fbits = keyvec ^ (neg & 0x7FFFFFFF)
    bnd_ref[...] = lax.bitcast_convert_type(fbits, jnp.float32)


def _classify_body(x_ref, bnd_ref, o_ref):
    xb = x_ref[...]
    acc = jnp.zeros(xb.shape, jnp.int32)
    for j in range(15):
        acc = acc + (xb > bnd_ref[0, j]).astype(jnp.int32)
    o_ref[...] = acc


def kernel(input):
    x = input
    assert x.shape == (_T,) and x.dtype == jnp.float32

    h1 = _hist1(x)
    lut16, meta = pl.pallas_call(
        _find1_body,
        out_shape=[jax.ShapeDtypeStruct((512, 128), jnp.int32),
                   jax.ShapeDtypeStruct((8, 128), jnp.int32)],
    )(h1.reshape(_WORKERS, 512, 128))

    h2, cnts, cand = _hist2(x, lut16.reshape(65536))
    lut2, meta2 = pl.pallas_call(
        _find2_body,
        out_shape=[jax.ShapeDtypeStruct((16, 256), jnp.int32),
                   jax.ShapeDtypeStruct((8, 128), jnp.int32)],
    )(h2.reshape(_WORKERS, 16, 256), meta)

    h3 = _hist3(cand, cnts, lut2.reshape(4096), x, lut16.reshape(65536))
    bnd = pl.pallas_call(
        _find3_body,
        out_shape=jax.ShapeDtypeStruct((1, 128), jnp.float32),
    )(h3.reshape(_WORKERS, 16, 256), meta2)

    rows, cols = 4096, 1024
    block = 512
    classes = pl.pallas_call(
        _classify_body,
        grid=(rows // block,),
        in_specs=[pl.BlockSpec((block, cols), lambda i: (i, 0)),
                  pl.BlockSpec((1, 128), lambda i: (0, 0))],
        out_specs=pl.BlockSpec((block, cols), lambda i: (i, 0)),
        out_shape=jax.ShapeDtypeStruct((rows, cols), jnp.int32),
    )(x.reshape(rows, cols), bnd)
    return classes.reshape(_T)


# speedup vs baseline: 1.0769x; 1.0222x over previous
import functools

import jax
import jax.numpy as jnp
import numpy as np
from jax import lax
from jax.experimental import pallas as pl
from jax.experimental.pallas import tpu as pltpu
from jax.experimental.pallas import tpu_sc as plsc

_NUM_CLASSES = 16
_IMBALANCE_RATIO = 4.0
_T = 4194304

_WORKERS = 32
_PER_W = _T // _WORKERS
_CH = 8192
_NCH = _PER_W // _CH
_L = 16
_CAP = 16384


def _rank_constants():
    r = _IMBALANCE_RATIO ** (1.0 / (_NUM_CLASSES - 1))
    p = [r**i for i in range(_NUM_CLASSES)]
    s = sum(p)
    p = [x / s for x in p]
    p.reverse()
    cp = np.cumsum(np.asarray(p[:-1], np.float32), dtype=np.float32)
    return [int(v) for v in (cp * np.float32(_T)).astype(np.int32)]

_RANKS = _rank_constants()

_sc_mesh = plsc.VectorSubcoreMesh(core_axis_name="c", subcore_axis_name="s")
_sc_params = pltpu.CompilerParams(needs_layout_passes=False)


def _zero_ref(ref, n):
    z = jnp.zeros((_L,), jnp.int32)

    @plsc.parallel_loop(0, n // _L, unroll=8)
    def body(i):
        ref[pl.ds(i * _L, _L)] = z


@functools.partial(
    pl.kernel,
    out_type=jax.ShapeDtypeStruct((_WORKERS, 65536), jnp.int32),
    mesh=_sc_mesh,
    scratch_types=[pltpu.VMEM((_CH,), jnp.float32),
                   pltpu.VMEM((65536,), jnp.int32)],
    compiler_params=_sc_params,
)
def _hist1(x_hbm, o_hbm, xb, tab):
    wid = lax.axis_index("s") * 2 + lax.axis_index("c")
    _zero_ref(tab, 65536)
    ones = jnp.ones((_L,), jnp.int32)
    base = wid * _PER_W

    def chunk(c, _):
        pltpu.sync_copy(x_hbm.at[pl.ds(base + c * _CH, _CH)], xb)

        @plsc.parallel_loop(0, _CH // _L, unroll=8)
        def vbody(i):
            r = plsc.bitcast(xb[pl.ds(i * _L, _L)], jnp.int32)
            bucket = lax.shift_right_logical(r, 16)
            plsc.addupdate_scatter(tab, [bucket], ones)

        return 0

    lax.fori_loop(0, _NCH, chunk, 0)
    pltpu.sync_copy(tab, o_hbm.at[wid])


@functools.partial(
    pl.kernel,
    out_type=[jax.ShapeDtypeStruct((_WORKERS, 4096), jnp.int32),
              jax.ShapeDtypeStruct((_WORKERS, 16), jnp.int32),
              jax.ShapeDtypeStruct((_WORKERS, _CAP), jnp.int32)],
    mesh=_sc_mesh,
    scratch_types=[pltpu.VMEM((_CH,), jnp.float32),
                   pltpu.VMEM((65536,), jnp.int32),
                   pltpu.VMEM((4096,), jnp.int32),
                   pltpu.VMEM((_CAP + _L,), jnp.int32),
                   pltpu.VMEM((_L,), jnp.int32)],
    compiler_params=_sc_params,
)
def _hist2(x_hbm, lut_hbm, o_hbm, cnt_hbm, cand_hbm, xb, lut, tab, cand,
           cntv):
    wid = lax.axis_index("s") * 2 + lax.axis_index("c")
    pltpu.sync_copy(lut_hbm, lut)
    _zero_ref(tab, 4096)
    ones = jnp.ones((_L,), jnp.int32)
    base = wid * _PER_W

    def chunk(c, cnt):
        pltpu.sync_copy(x_hbm.at[pl.ds(base + c * _CH, _CH)], xb)

        @plsc.parallel_loop(0, _CH // _L, unroll=8, carry=cnt)
        def vbody(i, cnt):
            r = plsc.bitcast(xb[pl.ds(i * _L, _L)], jnp.int32)
            lutv = plsc.load_gather(lut, [lax.shift_right_logical(r, 16)])
            m = lutv > 0
            mono16 = (r ^ lax.shift_right_arithmetic(r, 31)) & 0xFFFF
            w = lutv | mono16
            plsc.addupdate_scatter(tab, [lax.shift_right_logical(w, 8)],
                                   ones, mask=m)
            off = jnp.minimum(cnt, jnp.int32(_CAP))
            plsc.store_compressed(cand.at[pl.ds(off, _L)], w, mask=m)
            return cnt + jnp.sum(m.astype(jnp.int32))

        return vbody

    cnt = lax.fori_loop(0, _NCH, chunk, jnp.int32(0))
    pltpu.sync_copy(tab, o_hbm.at[wid])
    cntv[...] = jnp.full((_L,), cnt, jnp.int32)
    pltpu.sync_copy(cntv, cnt_hbm.at[wid])

    nch = (jnp.minimum(cnt, jnp.int32(_CAP)) + (_CH - 1)) // _CH

    def wout(c, _):
        pltpu.sync_copy(cand.at[pl.ds(c * _CH, _CH)],
                        cand_hbm.at[wid, pl.ds(c * _CH, _CH)])
        return 0

    lax.fori_loop(0, nch, wout, 0)


@functools.partial(
    pl.kernel,
    out_type=jax.ShapeDtypeStruct((_WORKERS, 4096), jnp.int32),
    mesh=_sc_mesh,
    scratch_types=[pltpu.VMEM((_CH,), jnp.int32),
                   pltpu.VMEM((_L,), jnp.int32),
                   pltpu.VMEM((4096,), jnp.int32),
                   pltpu.VMEM((4096,), jnp.int32),
                   pltpu.VMEM((_CH,), jnp.float32),
                   pltpu.VMEM((65536,), jnp.int32)],
    compiler_params=_sc_params,
)
def _hist3(cand_hbm, cnt_hbm, lut2_hbm, x_hbm, lut_hbm, o_hbm,
           cb, cntv, lut2, tab, xb, lut):
    wid = lax.axis_index("s") * 2 + lax.axis_index("c")
    pltpu.sync_copy(lut2_hbm, lut2)
    _zero_ref(tab, 4096)
    pltpu.sync_copy(cnt_hbm.at[wid], cntv)
    cnt = cntv[...][0]
    ones = jnp.ones((_L,), jnp.int32)
    lane = lax.broadcasted_iota(jnp.int32, (_L,), 0)

    def cand_path():
        nch = (cnt + (_CH - 1)) // _CH

        def chunk(c, _):
            pltpu.sync_copy(cand_hbm.at[wid, pl.ds(c * _CH, _CH)], cb)
            cbase = c * _CH

            @plsc.parallel_loop(0, _CH // _L, unroll=8)
            def vbody(i):
                w = cb[pl.ds(i * _L, _L)]
                lm = (cbase + i * _L + lane) < cnt
                v256 = plsc.load_gather(
                    lut2, [lax.shift_right_logical(w, 8) & 0xFFF])
                plsc.addupdate_scatter(tab, [v256 + (w & 0xFF)], ones, mask=lm)

            return 0

        lax.fori_loop(0, nch, chunk, 0)

    def full_path():
        pltpu.sync_copy(lut_hbm, lut)
        base = wid * _PER_W

        def chunk(c, _):
            pltpu.sync_copy(x_hbm.at[pl.ds(base + c * _CH, _CH)], xb)

            @plsc.parallel_loop(0, _CH // _L, unroll=8)
            def vbody(i):
                r = plsc.bitcast(xb[pl.ds(i * _L, _L)], jnp.int32)
                lutv = plsc.load_gather(
                    lut, [lax.shift_right_logical(r, 16)])
                m = lutv > 0
                mono16 = (r ^ lax.shift_right_arithmetic(r, 31)) & 0xFFFF
                w = lutv | mono16
                v256 = plsc.load_gather(
                    lut2, [lax.shift_right_logical(w, 8) & 0xFFF])
                plsc.addupdate_scatter(tab, [v256 + (w & 0xFF)], ones, mask=m)

            return 0

        lax.fori_loop(0, _NCH, chunk, 0)

    lax.cond(cnt <= _CAP, cand_path, full_path)
    pltpu.sync_copy(tab, o_hbm.at[wid])


def _tri_le(n):
    a = lax.broadcasted_iota(jnp.int32, (n, n), 0)
    b = lax.broadcasted_iota(jnp.int32, (n, n), 1)
    return (a <= b).astype(jnp.float32)


def _rev_mat(n):
    a = lax.broadcasted_iota(jnp.int32, (n, n), 0)
    b = lax.broadcasted_iota(jnp.int32, (n, n), 1)
    return (a + b == n - 1).astype(jnp.float32)


def _find1_body(hist_ref, lut_ref, meta_ref):
    hraw = jnp.sum(hist_ref[...].astype(jnp.float32), axis=0)
    neg = jnp.dot(jnp.dot(_rev_mat(256), hraw[256:, :],
                          preferred_element_type=jnp.float32,
                          precision=lax.Precision.HIGHEST),
                  _rev_mat(128), preferred_element_type=jnp.float32,
                  precision=lax.Precision.HIGHEST)
    h = jnp.concatenate([neg, hraw[:256, :]], axis=0)
    rowcum = jnp.dot(h, _tri_le(128), preferred_element_type=jnp.float32,
                     precision=lax.Precision.HIGHEST)
    srow = rowcum[:, 127:128]
    a = lax.broadcasted_iota(jnp.int32, (512, 512), 0)
    b = lax.broadcasted_iota(jnp.int32, (512, 512), 1)
    strict = (b < a).astype(jnp.float32)
    coarse = jnp.dot(strict, srow, preferred_element_type=jnp.float32,
                     precision=lax.Precision.HIGHEST)
    cum = rowcum + coarse

    bs, rs, firsts = [], [], []
    prev_b = None
    for k in _RANKS:
        le = (cum <= float(k)).astype(jnp.float32)
        bj = jnp.sum(le).astype(jnp.int32)
        cumexcl = jnp.max(cum * le)
        rj = jnp.int32(k) - cumexcl.astype(jnp.int32)
        bs.append(bj)
        rs.append(rj)
        firsts.append(jnp.int32(1) if prev_b is None
                      else (bj != prev_b).astype(jnp.int32))
        prev_b = bj
    uplus = []
    acc = jnp.int32(0)
    for f in firsts:
        acc = acc + f
        uplus.append(acc)

    gi = lax.broadcasted_iota(jnp.int32, (512, 128), 0)
    gj = lax.broadcasted_iota(jnp.int32, (512, 128), 1)
    g = gi * 128 + gj
    eq_any = jnp.zeros((512, 128), jnp.bool_)
    lev = jnp.zeros((512, 128), jnp.int32)
    for j in range(15):
        eq_any = eq_any | (g == bs[j])
        lev = lev + firsts[j] * (g >= bs[j]).astype(jnp.int32)
    lut_s = jnp.where(eq_any, lev * 65536, 0)
    flipped = jnp.dot(
        jnp.dot(_rev_mat(256), lut_s[:256, :].astype(jnp.float32),
                preferred_element_type=jnp.float32,
                precision=lax.Precision.HIGHEST),
        _rev_mat(128), preferred_element_type=jnp.float32,
        precision=lax.Precision.HIGHEST).astype(jnp.int32)
    lut_ref[...] = jnp.concatenate([lut_s[256:, :], flipped], axis=0)

    ri = lax.broadcasted_iota(jnp.int32, (8, 128), 0)
    ci = lax.broadcasted_iota(jnp.int32, (8, 128), 1)
    meta = jnp.zeros((8, 128), jnp.int32)
    for j in range(15):
        cj = (ci == j)
        meta = meta + jnp.where((ri == 0) & cj, rs[j], 0)
        meta = meta + jnp.where((ri == 1) & cj, uplus[j], 0)
        meta = meta + jnp.where((ri == 2) & cj, bs[j], 0)
    meta_ref[...] = meta


def _rows16(hist_ref):
    hsum = jnp.sum(hist_ref[...].astype(jnp.float32), axis=0, keepdims=True)
    rows = [lax.slice(hsum, (0, j * 256), (1, (j + 1) * 256))
            for j in range(16)]
    return jnp.concatenate(rows, axis=0)


def _find2_body(hist_ref, meta_ref, lut2_ref, meta2_ref):
    h = _rows16(hist_ref)
    cum = jnp.dot(h, _tri_le(256), preferred_element_type=jnp.float32,
                  precision=lax.Precision.HIGHEST)
    rowid = lax.broadcasted_iota(jnp.int32, (16, 256), 0)

    ss, r2s, firsts, pairids = [], [], [], []
    prev_pair = None
    for j in range(15):
        rj = meta_ref[0, j]
        uj = meta_ref[1, j]
        rm = (rowid == uj).astype(jnp.float32)
        cumsel = cum * rm
        le = (cumsel <= rj.astype(jnp.float32)).astype(jnp.float32) * rm
        sj = jnp.sum(le).astype(jnp.int32)
        cumexcl = jnp.max(cumsel * le)
        r2j = rj - cumexcl.astype(jnp.int32)
        pairj = uj * 256 + sj
        ss.append(sj)
        r2s.append(r2j)
        pairids.append(pairj)
        firsts.append(jnp.int32(1) if prev_pair is None
                      else (pairj != prev_pair).astype(jnp.int32))
        prev_pair = pairj
    vplus = []
    acc = jnp.int32(0)
    for f in firsts:
        acc = acc + f
        vplus.append(acc)

    colid = lax.broadcasted_iota(jnp.int32, (16, 256), 1)
    g = rowid * 256 + colid
    eq_any = jnp.zeros((16, 256), jnp.bool_)
    lev = jnp.zeros((16, 256), jnp.int32)
    for j in range(15):
        eq_any = eq_any | (g == pairids[j])
        lev = lev + firsts[j] * (g >= pairids[j]).astype(jnp.int32)
    lut2_ref[...] = jnp.where(eq_any, lev * 256, 0)

    ri = lax.broadcasted_iota(jnp.int32, (8, 128), 0)
    ci = lax.broadcasted_iota(jnp.int32, (8, 128), 1)
    meta2 = jnp.zeros((8, 128), jnp.int32)
    for j in range(15):
        cj = (ci == j)
        meta2 = meta2 + jnp.where((ri == 0) & cj, r2s[j], 0)
        meta2 = meta2 + jnp.where((ri == 1) & cj, vplus[j], 0)
        meta2 = meta2 + jnp.where((ri == 2) & cj, meta_ref[2, j], 0)
        meta2 = meta2 + jnp.where((ri == 3) & cj, ss[j], 0)
    meta2_ref[...] = meta2


def _find3_body(hist_ref, meta2_ref, bnd_ref):
    h = _rows16(hist_ref)
    cum = jnp.dot(h, _tri_le(256), preferred_element_type=jnp.float32,
                  precision=lax.Precision.HIGHEST)
    rowid = lax.broadcasted_iota(jnp.int32, (16, 256), 0)

    ci = lax.broadcasted_iota(jnp.int32, (1, 128), 1)
    keyvec = jnp.zeros((1, 128), jnp.int32)
    for j in range(15):
        r2j = meta2_ref[0, j]
        vj = meta2_ref[1, j]
        bj = meta2_ref[2, j]
        sj = meta2_ref[3, j]
        rm = (rowid == vj).astype(jnp.float32)
        cumsel = cum * rm
        le = (cumsel <= r2j.astype(jnp.float32)).astype(jnp.float32) * rm
        fj = jnp.sum(le).astype(jnp.int32)
        keyj = (bj - 32768) * 65536 + sj * 256 + fj
        keyvec = keyvec + jnp.where(ci == j, keyj, 0)
    neg = lax.shift_right_arithmetic(keyvec, 31)
    fbits = keyvec ^ (neg & 0x7FFFFFFF)
    bnd_ref[...] = lax.bitcast_convert_type(fbits, jnp.float32)


def _classify_body(x_ref, bnd_ref, o_ref):
    xb = x_ref[...]
    acc = jnp.zeros(xb.shape, jnp.int32)
    for j in range(15):
        acc = acc + (xb > bnd_ref[0, j]).astype(jnp.int32)
    o_ref[...] = acc


def kernel(input):
    x = input
    assert x.shape == (_T,) and x.dtype == jnp.float32

    h1 = _hist1(x)
    lut16, meta = pl.pallas_call(
        _find1_body,
        out_shape=[jax.ShapeDtypeStruct((512, 128), jnp.int32),
                   jax.ShapeDtypeStruct((8, 128), jnp.int32)],
    )(h1.reshape(_WORKERS, 512, 128))

    h2, cnts, cand = _hist2(x, lut16.reshape(65536))
    lut2, meta2 = pl.pallas_call(
        _find2_body,
        out_shape=[jax.ShapeDtypeStruct((16, 256), jnp.int32),
                   jax.ShapeDtypeStruct((8, 128), jnp.int32)],
    )(h2, meta)

    h3 = _hist3(cand, cnts, lut2.reshape(4096), x, lut16.reshape(65536))
    bnd = pl.pallas_call(
        _find3_body,
        out_shape=jax.ShapeDtypeStruct((1, 128), jnp.float32),
    )(h3, meta2)

    rows, cols = 4096, 1024
    block = 512
    classes = pl.pallas_call(
        _classify_body,
        grid=(rows // block,),
        in_specs=[pl.BlockSpec((block, cols), lambda i: (i, 0)),
                  pl.BlockSpec((1, 128), lambda i: (0, 0))],
        out_specs=pl.BlockSpec((block, cols), lambda i: (i, 0)),
        out_shape=jax.ShapeDtypeStruct((rows, cols), jnp.int32),
    )(x.reshape(rows, cols), bnd)
    return classes.reshape(_T)
